# baseline (speedup 1.0000x reference)
import functools

import jax
import jax.numpy as jnp
from jax import lax
from jax.experimental import pallas as pl
from jax.experimental.pallas import tpu as pltpu
from jax.experimental.pallas import tpu_sc as plsc

N = 50000
E = 800000
EP = 819200
NS = 51200
NF = 12
EF = 6
H = 64
G = 64

_f32 = jnp.float32


def _proj(hprev, sparts, w_in, b_in, wa, ba, wb):
    BN = 2000
    nsp = 0 if sparts is None else len(sparts)

    def body(*refs):
        h_ref = refs[0]
        s_refs = refs[1:1 + nsp]
        win_r, bin_r, wa_r, ba_r, wb_r = refs[1 + nsp:6 + nsp]
        hin_ref, a_ref, b_ref = refs[6 + nsp:]
        if nsp:
            s = jnp.concatenate([r[...] for r in s_refs], axis=1)
            h = jnp.maximum(h_ref[...] + s, 0.0)
        else:
            h = h_ref[...]
        hin = jnp.dot(h, win_r[...], preferred_element_type=_f32) + bin_r[...]
        hin_ref[...] = hin
        a_ref[...] = jnp.dot(hin, wa_r[...], preferred_element_type=_f32) \
            + ba_r[...]
        b_ref[...] = jnp.dot(hin, wb_r[...], preferred_element_type=_f32)

    hh_in = hprev.shape[1]
    ins = [hprev] + (sparts or []) + [w_in, b_in, wa, ba, wb]
    in_specs = (
        [pl.BlockSpec((BN, hh_in), lambda i: (i, 0))]
        + [pl.BlockSpec((BN, 16), lambda i: (i, 0))] * nsp
        + [pl.BlockSpec(w_in.shape, lambda i: (0, 0)),
           pl.BlockSpec((1, H), lambda i: (0, 0)),
           pl.BlockSpec((H, 2 * H), lambda i: (0, 0)),
           pl.BlockSpec((1, 2 * H), lambda i: (0, 0)),
           pl.BlockSpec((H, 2 * H), lambda i: (0, 0))]
    )
    return pl.pallas_call(
        body,
        grid=(N // BN,),
        in_specs=in_specs,
        out_specs=[pl.BlockSpec((BN, H), lambda i: (i, 0)),
                   pl.BlockSpec((BN, 2 * H), lambda i: (i, 0)),
                   pl.BlockSpec((BN, 2 * H), lambda i: (i, 0))],
        out_shape=[jax.ShapeDtypeStruct((N, H), _f32),
                   jax.ShapeDtypeStruct((N, 2 * H), _f32),
                   jax.ShapeDtypeStruct((N, 2 * H), _f32)],
    )(*ins)


def _edge_map(g, ea, we, nq):
    BE = 2048

    def body(g_ref, e_ref, we_r, *outs):
        y = g_ref[...] + jnp.dot(e_ref[...], we_r[...],
                                 preferred_element_type=_f32)
        m = jax.nn.sigmoid(y[:, :H]) * jax.nn.softplus(y[:, H:])
        for q in range(nq):
            outs[q][...] = m[:, q * 16:(q + 1) * 16]

    return pl.pallas_call(
        body,
        grid=(EP // BE,),
        in_specs=[pl.BlockSpec((BE, 2 * H), lambda i: (i, 0)),
                  pl.BlockSpec((BE, EF), lambda i: (i, 0)),
                  pl.BlockSpec((EF, 2 * H), lambda i: (0, 0))],
        out_specs=[pl.BlockSpec((BE, 16), lambda i: (i, 0))] * nq,
        out_shape=[jax.ShapeDtypeStruct((EP, 16), _f32)] * nq,
    )(g, ea, we)


def _final(h2, sparts, batch2, wh1, bh1, wh2, bh2):
    BN = 2000
    nblk = N // BN

    def body(h_ref, s0_ref, s1_ref, s2_ref, s3_ref, b_ref,
             wh1_r, bh1_r, wh2_r, bh2_r, o_ref, acc_s, acc_c):
        i = pl.program_id(0)

        @pl.when(i == 0)
        def _():
            acc_s[...] = jnp.zeros_like(acc_s)
            acc_c[...] = jnp.zeros_like(acc_c)

        s = jnp.concatenate([s0_ref[...], s1_ref[...],
                             s2_ref[...], s3_ref[...]], axis=1)
        h3 = jnp.maximum(h_ref[...] + s, 0.0)
        seg = b_ref[...]
        oh = (seg == lax.broadcasted_iota(jnp.int32, (BN, G), 1)).astype(_f32)
        acc_s[...] += lax.dot_general(oh, h3, (((0,), (0,)), ((), ())),
                                      preferred_element_type=_f32)
        acc_c[...] += lax.dot_general(oh, jnp.ones((BN, 1), _f32),
                                      (((0,), (0,)), ((), ())),
                                      preferred_element_type=_f32)

        @pl.when(i == nblk - 1)
        def _():
            pooled = acc_s[...] / jnp.maximum(acc_c[...], 1.0)
            t = jnp.dot(pooled, wh1_r[...], preferred_element_type=_f32)
            t = jnp.maximum(t + bh1_r[...], 0.0)
            o_ref[...] = (jnp.dot(t, wh2_r[...], preferred_element_type=_f32)
                          + bh2_r[...])

    return pl.pallas_call(
        body,
        grid=(nblk,),
        in_specs=[pl.BlockSpec((BN, H), lambda i: (i, 0))]
        + [pl.BlockSpec((BN, 16), lambda i: (i, 0))] * 4
        + [pl.BlockSpec((BN, 1), lambda i: (i, 0)),
           pl.BlockSpec((H, H), lambda i: (0, 0)),
           pl.BlockSpec((1, H), lambda i: (0, 0)),
           pl.BlockSpec((H, 1), lambda i: (0, 0)),
           pl.BlockSpec((1, 1), lambda i: (0, 0))],
        out_specs=pl.BlockSpec((G, 1), lambda i: (0, 0)),
        out_shape=jax.ShapeDtypeStruct((G, 1), _f32),
        scratch_shapes=[pltpu.VMEM((G, H), _f32),
                        pltpu.VMEM((G, 1), _f32)],
    )(h2, *sparts, batch2, wh1, bh1, wh2, bh2)


_MESH = plsc.VectorSubcoreMesh(core_axis_name="c", subcore_axis_name="s")


def _sc_gather(a_t, b_t, dst2, src2):
    NB = EP // 128 // 32
    DT = 2 * H

    @functools.partial(
        pl.kernel,
        mesh=_MESH,
        out_type=jax.ShapeDtypeStruct((EP, DT), _f32),
        scratch_types=[
            pltpu.VMEM((NB, 128), jnp.int32),
            pltpu.VMEM((NB, 128), jnp.int32),
            pltpu.VMEM((128, DT), _f32),
            pltpu.VMEM((128, DT), _f32),
            pltpu.SemaphoreType.DMA,
            pltpu.SemaphoreType.DMA,
        ],
    )
    def k(a_h, b_h, d_h, s_h, g_h, idxd, idxs, bufa, bufb, sem1, sem2):
        wid = lax.axis_index("s") * 2 + lax.axis_index("c")
        cbase = wid * NB
        pltpu.sync_copy(d_h.at[pl.ds(cbase, NB)], idxd)
        pltpu.sync_copy(s_h.at[pl.ds(cbase, NB)], idxs)

        def chunk(j, carry):
            ca = pltpu.async_copy(a_h.at[idxd.at[j]], bufa, sem1)
            cb = pltpu.async_copy(b_h.at[idxs.at[j]], bufb, sem2)
            ca.wait()
            cb.wait()

            def row(r, c2):
                for c in range(DT // 16):
                    bufa[r, pl.ds(c * 16, 16)] = (
                        bufa[r, pl.ds(c * 16, 16)]
                        + bufb[r, pl.ds(c * 16, 16)])
                return c2

            lax.fori_loop(0, 128, row, 0)
            pltpu.sync_copy(bufa, g_h.at[pl.ds((cbase + j) * 128, 128)])
            return carry

        lax.fori_loop(0, NB, chunk, 0)

    return k(a_t, b_t, dst2, src2)


def _sc_scatter(m_a, m_b, dsts2):
    NBT = EP // 128 // 16
    ROWS_T = NS // 16
    NZ = ROWS_T // 128

    @functools.partial(
        pl.kernel,
        mesh=_MESH,
        out_type=[jax.ShapeDtypeStruct((NS // 8, 128), _f32),
                  jax.ShapeDtypeStruct((NS // 8, 128), _f32)],
        scratch_types=[
            pltpu.VMEM((8, 128), jnp.int32),
            pltpu.VMEM((16, 128), _f32),
            pltpu.VMEM((16, 16), _f32),
            pltpu.VMEM((16, 16), _f32),
            pltpu.VMEM((16, 16), _f32),
            pltpu.VMEM((16, 128), _f32),
            pltpu.VMEM((16,), jnp.int32),
            pltpu.VMEM_SHARED((NS, 16), _f32),
        ],
    )
    def k(ma_h, mb_h, d_h, sa_out, sb_out, idx, mpack, mbuf16, zstage, wbuf16,
          wpack, idx16, shared):
        c = lax.axis_index("c")
        t = lax.axis_index("s")

        ramp = lax.iota(jnp.int32, 16)

        def zeros_row(r, carry):
            zstage[r, pl.ds(0, 16)] = jnp.zeros((16,), _f32)
            return carry

        lax.fori_loop(0, 16, zeros_row, 0)

        def zchunk(z, carry):
            idx16[pl.ds(0, 16)] = ramp + (t * ROWS_T + z * 16)
            pltpu.sync_copy(zstage, shared.at[idx16])
            return carry

        lax.fori_loop(0, ROWS_T // 16, zchunk, 0)
        plsc.subcore_barrier()

        def block(ib, carry):
            pltpu.sync_copy(d_h.at[pl.ds(t * NBT + ib * 8, 8)], idx)

            def chunk(j, c2):
                moff = pl.multiple_of((t * NBT + ib * 8 + j) * 16, 16)

                @pl.when(c == 0)
                def _():
                    pltpu.sync_copy(ma_h.at[pl.ds(moff, 16)], mpack)

                @pl.when(c == 1)
                def _():
                    pltpu.sync_copy(mb_h.at[pl.ds(moff, 16)], mpack)

                for k16 in range(8):
                    for i in range(16):
                        e = k16 * 16 + i
                        mbuf16[i, pl.ds(0, 16)] = \
                            mpack[e // 8, pl.ds((e % 8) * 16, 16)]
                    idx16[pl.ds(0, 16)] = idx[j, pl.ds(k16 * 16, 16)]
                    pltpu.sync_copy(mbuf16, shared.at[idx16], add=True)
                return c2

            lax.fori_loop(0, 8, chunk, 0)
            return carry

        lax.fori_loop(0, NBT // 8, block, 0)
        plsc.subcore_barrier()

        def wchunk(z, carry):
            base = t * ROWS_T + z * 128
            for k16 in range(8):
                idx16[pl.ds(0, 16)] = ramp + (base + k16 * 16)
                pltpu.sync_copy(shared.at[idx16], wbuf16)
                for i in range(16):
                    r = k16 * 16 + i
                    wpack[r // 8, pl.ds((r % 8) * 16, 16)] = \
                        wbuf16[i, pl.ds(0, 16)]
            woff = pl.multiple_of(base // 8, 16)

            @pl.when(c == 0)
            def _():
                pltpu.sync_copy(wpack, sa_out.at[pl.ds(woff, 16)])

            @pl.when(c == 1)
            def _():
                pltpu.sync_copy(wpack, sb_out.at[pl.ds(woff, 16)])

            return carry

        lax.fori_loop(0, NZ, wchunk, 0)

    sa_p, sb_p = k(m_a.reshape(EP // 8, 128), m_b.reshape(EP // 8, 128), dsts2)
    return sa_p.reshape(NS, 16), sb_p.reshape(NS, 16)


def kernel(x, edge_index, edge_attr, batch, Wf1, bf1, Ws1, bs1, W_in, b_in,
           Wf2, bf2, Ws2, bs2, Wf3, bf3, Ws3, bs3, Wh1, bh1, Wh2, bh2):
    src = edge_index[0]
    dst = edge_index[1]
    pad = EP - E

    def _xla_scatter(m_a, m_b, d2):
        di = d2.reshape(-1)
        sa = jnp.zeros((NS, 16), _f32).at[di].add(m_a)
        sb = jnp.zeros((NS, 16), _f32).at[di].add(m_b)
        return sa, sb

    zpad = jnp.zeros((pad,), jnp.int32)
    dst2g = jnp.concatenate([dst, zpad]).reshape(EP // 128, 128)
    src2g = jnp.concatenate([src, zpad]).reshape(EP // 128, 128)
    dst2s = jnp.concatenate([dst, jnp.full((pad,), N, jnp.int32)]
                            ).reshape(EP // 128, 128)
    ea_p = jnp.concatenate([edge_attr, jnp.zeros((pad, EF), _f32)], axis=0)

    def padh(w):
        return jnp.pad(w, ((0, 0), (0, H - NF)))

    def pad2(w):
        return jnp.pad(w, ((0, H - NF), (0, H - NF)))

    wa1 = jnp.concatenate([pad2(Wf1[:NF]), pad2(Ws1[:NF])], axis=1)
    ba1 = jnp.concatenate([jnp.pad(bf1, (0, H - NF)),
                           jnp.pad(bs1, (0, H - NF))])[None, :]
    wb1 = jnp.concatenate([pad2(Wf1[NF:2 * NF]), pad2(Ws1[NF:2 * NF])], axis=1)
    we1 = jnp.concatenate([padh(Wf1[2 * NF:]), padh(Ws1[2 * NF:])], axis=1)

    xpad = jnp.pad(x, ((0, 0), (0, H - NF)))
    eye = jnp.eye(H, dtype=_f32)
    zb = jnp.zeros((1, H), _f32)
    _, a1, b1 = _proj(xpad, None, eye, zb, wa1, ba1, wb1)
    g1 = _sc_gather(a1, b1, dst2g, src2g)
    (m1,) = _edge_map(g1, ea_p, we1, 1)
    s1a, s1b = _xla_scatter(m1, m1, dst2s)

    win_p = jnp.pad(W_in, ((0, H - NF), (0, 0)))
    wa2 = jnp.concatenate([Wf2[:H], Ws2[:H]], axis=1)
    ba2 = jnp.concatenate([bf2, bs2])[None, :]
    wb2 = jnp.concatenate([Wf2[H:2 * H], Ws2[H:2 * H]], axis=1)
    we2 = jnp.concatenate([Wf2[2 * H:], Ws2[2 * H:]], axis=1)
    hin, a2, b2 = _proj(xpad, [s1a, s1b, s1a, s1b], win_p, b_in[None, :],
                        wa2, ba2, wb2)
    g2 = _sc_gather(a2, b2, dst2g, src2g)
    m2 = _edge_map(g2, ea_p, we2, 4)
    s2q0a, s2q0b = _xla_scatter(m2[0], m2[2], dst2s)
    s2q1a, s2q1b = _xla_scatter(m2[1], m2[3], dst2s)
    s2parts = [s2q0a, s2q1a, s2q0b, s2q1b]

    wa3 = jnp.concatenate([Wf3[:H], Ws3[:H]], axis=1)
    ba3 = jnp.concatenate([bf3, bs3])[None, :]
    wb3 = jnp.concatenate([Wf3[H:2 * H], Ws3[H:2 * H]], axis=1)
    we3 = jnp.concatenate([Wf3[2 * H:], Ws3[2 * H:]], axis=1)
    h2, a3, b3 = _proj(hin, s2parts, eye, zb, wa3, ba3, wb3)
    g3 = _sc_gather(a3, b3, dst2g, src2g)
    m3 = _edge_map(g3, ea_p, we3, 4)
    s3q0a, s3q0b = _xla_scatter(m3[0], m3[2], dst2s)
    s3q1a, s3q1b = _xla_scatter(m3[1], m3[3], dst2s)
    s3parts = [s3q0a, s3q1a, s3q0b, s3q1b]

    batch2 = batch.reshape(N, 1)
    return _final(h2, s3parts, batch2, Wh1, bh1[None, :], Wh2, bh2[None, :])
